# Initial kernel scaffold; baseline (speedup 1.0000x reference)
import jax
import jax.numpy as jnp
from jax.experimental import pallas as pl


def kernel(x, edge_index, batch, W_in, b_in, Wc0, bc0, g0, be0, Wc1, bc1, g1, be1, Wm1, bm1, Wm2, bm2):
    raise NotImplementedError("write your pallas kernel here")


# speedup vs baseline: 20.3856x; 20.3856x over previous
import functools

import jax
import jax.numpy as jnp
from jax import lax
from jax.experimental import pallas as pl
from jax.experimental.pallas import tpu as pltpu
from jax.experimental.pallas import tpu_sc as plsc

N = 50000
E = 800000
H = 64
HH = H // 2
NG = 64
EPS = 1e-5

CH = 128
NCORES = 2
NTILES = 16
NP = N + 8
ACC_ROWS = 50048
ZROWS = ACC_ROWS // NTILES
DACC = 51200
DZ = DACC // NTILES

EP = 819200
SBR = 8
KIO = 2
NSB = EP // (NTILES * CH * SBR)
RPT_AGG = EP // (NTILES * CH)
KF_DEG = 8
NB_DEG = EP // (NCORES * NTILES * CH * KF_DEG)
RPT_DEG = EP // (NCORES * NTILES * CH)

BN = 2048
NBLK = (N + BN - 1) // BN

_MESH = plsc.VectorSubcoreMesh(core_axis_name="c", subcore_axis_name="s")


@functools.partial(
    pl.kernel,
    out_type=jax.ShapeDtypeStruct((NCORES * DACC,), jnp.float32),
    mesh=_MESH,
    scratch_types=[
        pltpu.VMEM((KF_DEG, CH), jnp.int32),
        pltpu.VMEM((KF_DEG, CH), jnp.int32),
        pltpu.VMEM((CH,), jnp.float32),
        pltpu.VMEM_SHARED((DACC,), jnp.float32),
        pltpu.SemaphoreType.DMA,
        pltpu.SemaphoreType.DMA,
    ],
)
def _sc_deg(dstp2, zeros1, ones_h, out, didx0, didx1, ones_v, acc,
            isem, ssem):
    c = lax.axis_index("c")
    s = lax.axis_index("s")
    didx = [didx0, didx1]
    rbase = (c * NTILES + s) * RPT_DEG

    def idx_fire(g, b):
        pltpu.async_copy(dstp2.at[pl.ds(rbase + g * KF_DEG, KF_DEG)],
                         didx[b], isem)

    def idx_drain(g, b):
        pltpu.make_async_copy(dstp2.at[pl.ds(rbase + g * KF_DEG, KF_DEG)],
                              didx[b], isem).wait()

    def scat_fire(b):
        for k in range(KF_DEG):
            pltpu.async_copy(ones_v, acc.at[didx[b].at[k]], ssem, add=True)

    def scat_drain(b):
        for k in range(KF_DEG):
            pltpu.make_async_copy(ones_v, acc.at[didx[b].at[k]], ssem).wait()

    idx_fire(0, 0)
    pltpu.sync_copy(zeros1, acc.at[pl.ds(s * DZ, DZ)])
    pltpu.sync_copy(ones_h, ones_v)
    plsc.subcore_barrier()

    def block(g, b):
        g = jnp.int32(g)
        nb = 1 - b
        idx_drain(g, b)

        @pl.when(g >= 1)
        def _():
            scat_drain(nb)

        @pl.when(g + 1 < NB_DEG)
        def _():
            idx_fire(g + 1, nb)

        scat_fire(b)

    def pair(g2, carry):
        block(2 * g2, 0)
        block(2 * g2 + 1, 1)
        return carry

    lax.fori_loop(0, NB_DEG // 2, pair, 0)
    if NB_DEG % 2:
        block(NB_DEG - 1, 0)
        scat_drain(0)
    else:
        scat_drain(1)
    plsc.subcore_barrier()
    pltpu.sync_copy(acc.at[pl.ds(s * DZ, DZ)],
                    out.at[pl.ds(c * DACC + s * DZ, DZ)])


@functools.partial(
    pl.kernel,
    out_type=jax.ShapeDtypeStruct((NCORES * ACC_ROWS, HH), jnp.float32),
    mesh=_MESH,
    compiler_params=pltpu.CompilerParams(use_tc_tiling_on_sc=False),
    scratch_types=[
        pltpu.VMEM((SBR, CH), jnp.int32),
        pltpu.VMEM((SBR, CH), jnp.int32),
        pltpu.VMEM((SBR, CH), jnp.int32),
        pltpu.VMEM((SBR, CH), jnp.int32),
        pltpu.VMEM((KIO, CH, HH), jnp.float32),
        pltpu.VMEM((KIO, CH, HH), jnp.float32),
        pltpu.VMEM((KIO, CH, HH), jnp.float32),
        pltpu.VMEM_SHARED((ACC_ROWS, HH), jnp.float32),
        pltpu.SemaphoreType.DMA,
        pltpu.SemaphoreType.DMA,
        pltpu.SemaphoreType.DMA,
        pltpu.SemaphoreType.DMA,
        pltpu.SemaphoreType.DMA,
    ],
)
def _sc_agg(utab, srcp2, dstp2, zeros2, out, sidx0, sidx1, didx0, didx1,
            rows0, rows1, rows2, acc, isem, gsem, ssem0, ssem1, ssem2):
    c = lax.axis_index("c")
    s = lax.axis_index("s")
    sidx = [sidx0, sidx1]
    didx = [didx0, didx1]
    rows = [rows0, rows1, rows2]
    ssem = [ssem0, ssem1, ssem2]
    rbase = s * RPT_AGG
    csplat = jnp.full((16,), c * NP, dtype=jnp.int32)

    def idx_fire(g, b):
        pltpu.async_copy(srcp2.at[pl.ds(rbase + g * SBR, SBR)], sidx[b], isem)
        pltpu.async_copy(dstp2.at[pl.ds(rbase + g * SBR, SBR)], didx[b], isem)

    def idx_drain(g, b):
        pltpu.make_async_copy(srcp2.at[pl.ds(rbase + g * SBR, SBR)],
                              sidx[b], isem).wait()
        pltpu.make_async_copy(dstp2.at[pl.ds(rbase + g * SBR, SBR)],
                              didx[b], isem).wait()

    idx_fire(0, 0)
    pltpu.sync_copy(zeros2, acc.at[pl.ds(s * ZROWS, ZROWS)])
    plsc.subcore_barrier()

    def superblock(S, b):
        nb = 1 - b
        idx_drain(S, b)
        for k in range(SBR):
            for j in range(CH // 16):
                sl = pl.ds(j * 16, 16)
                sidx[b][k, sl] = sidx[b][k, sl] + csplat

        def gfire(t, slot):
            return [pltpu.async_copy(utab.at[sidx[b].at[KIO * t + k]],
                                     rows[slot].at[k], gsem)
                    for k in range(KIO)]

        def sfire(t, slot):
            for k in range(KIO):
                pltpu.async_copy(rows[slot].at[k],
                                 acc.at[didx[b].at[KIO * t + k]],
                                 ssem[slot], add=True)

        def sdrain(slot):
            for k in range(KIO):
                pltpu.make_async_copy(rows[slot].at[k], acc.at[didx0.at[k]],
                                      ssem[slot]).wait()

        @pl.when(S >= 1)
        def _():
            sdrain(0)
        g0 = gfire(0, 0)
        g1 = gfire(1, 1)

        for d in g0:
            d.wait()
        sfire(0, 0)
        g2 = gfire(2, 2)

        @pl.when(S + 1 < NSB)
        def _():
            idx_fire(S + 1, nb)

        for d in g1:
            d.wait()
        sfire(1, 1)

        sdrain(0)
        g3 = gfire(3, 0)

        for d in g2:
            d.wait()
        sfire(2, 2)
        sdrain(1)

        for d in g3:
            d.wait()
        sfire(3, 0)
        sdrain(2)

    def pair(S2, carry):
        superblock(2 * S2, 0)
        superblock(2 * S2 + 1, 1)
        return carry

    lax.fori_loop(0, NSB // 2, pair, 0)
    for k in range(KIO):
        pltpu.make_async_copy(rows[0].at[k], acc.at[didx0.at[k]],
                              ssem[0]).wait()
    plsc.subcore_barrier()
    pltpu.sync_copy(acc.at[pl.ds(s * ZROWS, ZROWS)],
                    out.at[pl.ds(c * ACC_ROWS + s * ZROWS, ZROWS)])


def _dotT(a, b):
    return lax.dot_general(a, b, (((1,), (1,)), ((), ())),
                           preferred_element_type=jnp.float32)


def _valid_rows():
    ids = (pl.program_id(0) * BN
           + lax.broadcasted_iota(jnp.int32, (BN, 1), 0))
    return ids < N


def _tca_body(x_ref, dp_ref, win_ref, bin_ref, wc0_ref, u0_ref, dis_ref):
    deg = dp_ref[0] + dp_ref[1] + 1.0
    dis = lax.rsqrt(deg)
    h = jnp.maximum(_dotT(x_ref[...], win_ref[...]) + bin_ref[...], 0.0)
    u0 = _dotT(h, wc0_ref[...]) * dis[:, None]
    u0_ref[...] = jnp.where(_valid_rows(), u0, 0.0)
    dis_ref[...] = dis[None, :]


def _tcb_body(s0_ref, u0_ref, dis_ref, bc0_ref, g0_ref, be0_ref, wc1_ref,
              h1_ref, u1_ref):
    dis = dis_ref[...][0][:, None]
    agg = dis * (s0_ref[...] + u0_ref[...]) + bc0_ref[...]
    gs = g0_ref[...] * (1.0 / jnp.sqrt(1.0 + EPS))
    h1 = jnp.maximum(agg * gs + be0_ref[...], 0.0)
    u1 = _dotT(h1, wc1_ref[...]) * dis
    h1_ref[...] = h1
    u1_ref[...] = jnp.where(_valid_rows(), u1, 0.0)


def _tcc_body(s1_ref, u1_ref, dis_ref, h1_ref, batch_ref, bc1_ref, g1_ref,
              be1_ref, wm1_ref, bm1_ref, wm2_ref, bm2_ref, out_ref, ps, cs):
    i = pl.program_id(0)

    @pl.when(i == 0)
    def _init():
        ps[...] = jnp.zeros_like(ps)
        cs[...] = jnp.zeros_like(cs)

    dis = dis_ref[...][0][:, None]
    agg = dis * (s1_ref[...] + u1_ref[...]) + bc1_ref[...]
    gs = g1_ref[...] * (1.0 / jnp.sqrt(1.0 + EPS))
    h2 = h1_ref[...] + jnp.maximum(agg * gs + be1_ref[...], 0.0)

    valid = _valid_rows()
    h2m = jnp.where(valid, h2, 0.0)
    cols = lax.broadcasted_iota(jnp.int32, (1, NG), 1)
    oh = jnp.where((batch_ref[...][0][:, None] == cols) & valid, 1.0, 0.0)
    onescol = jnp.where(valid, 1.0, 0.0)
    ps[...] += lax.dot_general(oh, h2m, (((0,), (0,)), ((), ())),
                               preferred_element_type=jnp.float32)
    cs[...] += lax.dot_general(oh, onescol, (((0,), (0,)), ((), ())),
                               preferred_element_type=jnp.float32)

    @pl.when(i == NBLK - 1)
    def _final():
        pooled = ps[...] / jnp.maximum(cs[...], 1.0)
        z = jnp.maximum(_dotT(pooled, wm1_ref[...]) + bm1_ref[...], 0.0)
        out_ref[...] = (jnp.sum(z * wm2_ref[...], axis=1, keepdims=True)
                        + bm2_ref[...])


def _full(shape):
    return pl.BlockSpec(shape, lambda i: tuple(0 for _ in shape))


_tca = pl.pallas_call(
    _tca_body,
    grid=(NBLK,),
    in_specs=[
        pl.BlockSpec((BN, 7), lambda i: (i, 0)),
        pl.BlockSpec((2, BN), lambda i: (0, i)),
        _full((H, 7)),
        _full((1, H)),
        _full((H, H)),
    ],
    out_specs=[
        pl.BlockSpec((BN, H), lambda i: (i, 0)),
        pl.BlockSpec((1, BN), lambda i: (0, i)),
    ],
    out_shape=[
        jax.ShapeDtypeStruct((NP, H), jnp.float32),
        jax.ShapeDtypeStruct((1, N), jnp.float32),
    ],
)

_tcb = pl.pallas_call(
    _tcb_body,
    grid=(NBLK,),
    in_specs=[
        pl.BlockSpec((BN, H), lambda i: (i, 0)),
        pl.BlockSpec((BN, H), lambda i: (i, 0)),
        pl.BlockSpec((1, BN), lambda i: (0, i)),
        _full((1, H)),
        _full((1, H)),
        _full((1, H)),
        _full((H, H)),
    ],
    out_specs=[
        pl.BlockSpec((BN, H), lambda i: (i, 0)),
        pl.BlockSpec((BN, H), lambda i: (i, 0)),
    ],
    out_shape=[
        jax.ShapeDtypeStruct((N, H), jnp.float32),
        jax.ShapeDtypeStruct((NP, H), jnp.float32),
    ],
)

_tcc = pl.pallas_call(
    _tcc_body,
    grid=(NBLK,),
    in_specs=[
        pl.BlockSpec((BN, H), lambda i: (i, 0)),
        pl.BlockSpec((BN, H), lambda i: (i, 0)),
        pl.BlockSpec((1, BN), lambda i: (0, i)),
        pl.BlockSpec((BN, H), lambda i: (i, 0)),
        pl.BlockSpec((1, BN), lambda i: (0, i)),
        _full((1, H)),
        _full((1, H)),
        _full((1, H)),
        _full((HH, H)),
        _full((1, HH)),
        _full((1, HH)),
        _full((1, 1)),
    ],
    out_specs=_full((NG, 1)),
    out_shape=jax.ShapeDtypeStruct((NG, 1), jnp.float32),
    scratch_shapes=[
        pltpu.VMEM((NG, NG), jnp.float32),
        pltpu.VMEM((NG, 1), jnp.float32),
    ],
)


def kernel(x, edge_index, batch, W_in, b_in, Wc0, bc0, g0, be0,
           Wc1, bc1, g1, be1, Wm1, bm1, Wm2, bm2):
    src = edge_index[0]
    dst = edge_index[1]
    pad = EP - E
    srcp = jnp.concatenate([src, jnp.full((pad,), N, jnp.int32)])
    dstp = jnp.concatenate([dst, jnp.full((pad,), N, jnp.int32)])
    srcp = srcp.reshape(EP // CH, CH)
    dstp = dstp.reshape(EP // CH, CH)
    zeros1 = jnp.zeros((DZ,), jnp.float32)
    zeros2 = jnp.zeros((ZROWS, HH), jnp.float32)
    ones_h = jnp.ones((CH,), jnp.float32)

    degp = _sc_deg(dstp, zeros1, ones_h)
    dp = degp.reshape(NCORES, DACC)

    def to_planar(u):
        return u.reshape(NP, 2, HH).transpose(1, 0, 2).reshape(2 * NP, HH)

    def from_planar(sp):
        return (sp.reshape(2, ACC_ROWS, HH).transpose(1, 0, 2)
                .reshape(ACC_ROWS, H))

    u0, dis = _tca(x, dp, W_in, b_in.reshape(1, H), Wc0)
    s0 = from_planar(_sc_agg(to_planar(u0), srcp, dstp, zeros2))
    h1, u1 = _tcb(s0, u0, dis,
                  bc0.reshape(1, H), g0.reshape(1, H), be0.reshape(1, H), Wc1)
    s1 = from_planar(_sc_agg(to_planar(u1), srcp, dstp, zeros2))
    out = _tcc(s1, u1, dis, h1,
               batch.reshape(1, N), bc1.reshape(1, H), g1.reshape(1, H),
               be1.reshape(1, H), Wm1, bm1.reshape(1, HH),
               Wm2, bm2.reshape(1, 1))
    return out
